# baseline (speedup 1.0000x reference)
import functools

import jax
import jax.numpy as jnp
from jax import lax
from jax.experimental import pallas as pl
from jax.experimental.pallas import tpu as pltpu
from jax.experimental.pallas import tpu_sc as plsc

_B = 4
_T = 2048
_D = 128
_NPROMPT = 8

_INFO = plsc.get_sparse_core_info()
_NC = _INFO.num_cores
_NS = _INFO.num_subcores
_NW = _NC * _NS
_TPW = _T // _NW
_LANES = 16
_CHUNKS = _D // _LANES


def _body(idx_hbm, wte_hbm, wpe_hbm, prm_hbm, out_hbm,
          idx_v, rows_v, wpe_v, prm_v,
          sem_g0, sem_g1, sem_g2, sem_g3,
          sem_h0, sem_h1, sem_h2, sem_h3, sem_w, sem_p, sem_st):
    gsems = (sem_g0, sem_g1, sem_g2, sem_g3)
    hsems = (sem_h0, sem_h1, sem_h2, sem_h3)
    wid = lax.axis_index("s") * _NC + lax.axis_index("c")
    t0 = wid * _TPW

    idx_cps = [pltpu.async_copy(idx_hbm.at[b, pl.ds(t0, _TPW)], idx_v.at[b],
                                gsems[b]) for b in range(_B)]
    wpe_cp = pltpu.async_copy(wpe_hbm.at[pl.ds(t0, _TPW)], wpe_v, sem_w)

    @pl.when(wid < _B)
    def _():
        pltpu.async_copy(prm_hbm, prm_v, sem_p).wait()
        pltpu.async_copy(prm_v, out_hbm.at[wid, pl.ds(0, _NPROMPT)],
                         sem_st)

    half = _TPW // 2
    gathers = []
    for b in range(_B):
        idx_cps[b].wait()
        gathers.append((
            pltpu.async_copy(wte_hbm.at[idx_v.at[b, pl.ds(0, half)]],
                             rows_v.at[b].at[pl.ds(0, half)], gsems[b]),
            pltpu.async_copy(wte_hbm.at[idx_v.at[b, pl.ds(half, half)]],
                             rows_v.at[b].at[pl.ds(half, half)], hsems[b]),
        ))
    wpe_cp.wait()

    stores = []
    for b in range(_B):
        buf = rows_v.at[b]

        def _add_row(i, _):
            for j in range(_CHUNKS):
                sl = pl.ds(j * _LANES, _LANES)
                plsc.addupdate(buf.at[i, sl], wpe_v[i, sl])
            return _

        gathers[b][0].wait()
        lax.fori_loop(0, half, _add_row, None)
        stores.append(pltpu.async_copy(
            buf.at[pl.ds(0, half)],
            out_hbm.at[b, pl.ds(_NPROMPT + t0, half)], sem_st))
        gathers[b][1].wait()
        lax.fori_loop(half, _TPW, _add_row, None)
        stores.append(pltpu.async_copy(
            buf.at[pl.ds(half, half)],
            out_hbm.at[b, pl.ds(_NPROMPT + t0 + half, half)], sem_st))

    for cp in stores:
        cp.wait()

    @pl.when(wid < _B)
    def _():
        pltpu.make_async_copy(prm_v, out_hbm.at[wid, pl.ds(0, _NPROMPT)],
                              sem_st).wait()


@functools.partial(jax.jit, static_argnames=())
def kernel(idx, wte, wpe, reasoning_prompts):
    b, t = idx.shape
    assert (b, t) == (_B, _T)
    mesh = plsc.VectorSubcoreMesh(core_axis_name="c", subcore_axis_name="s")
    run = pl.kernel(
        _body,
        out_type=jax.ShapeDtypeStruct((_B, _NPROMPT + _T, _D), jnp.float32),
        mesh=mesh,
        scratch_types=[
            pltpu.VMEM((_B, _TPW), jnp.int32),
            pltpu.VMEM((_B, _TPW, _D), jnp.float32),
            pltpu.VMEM((_TPW, _D), jnp.float32),
            pltpu.VMEM((_NPROMPT, _D), jnp.float32),
            pltpu.SemaphoreType.DMA,
            pltpu.SemaphoreType.DMA,
            pltpu.SemaphoreType.DMA,
            pltpu.SemaphoreType.DMA,
            pltpu.SemaphoreType.DMA,
            pltpu.SemaphoreType.DMA,
            pltpu.SemaphoreType.DMA,
            pltpu.SemaphoreType.DMA,
            pltpu.SemaphoreType.DMA,
            pltpu.SemaphoreType.DMA,
            pltpu.SemaphoreType.DMA,
        ],
    )
    return run(idx.astype(jnp.int32), wte, wpe, reasoning_prompts)


# speedup vs baseline: 1.0022x; 1.0022x over previous
import functools

import jax
import jax.numpy as jnp
from jax import lax
from jax.experimental import pallas as pl
from jax.experimental.pallas import tpu as pltpu
from jax.experimental.pallas import tpu_sc as plsc

_B = 4
_T = 2048
_D = 128
_NPROMPT = 8

_INFO = plsc.get_sparse_core_info()
_NC = _INFO.num_cores
_NS = _INFO.num_subcores
_NW = _NC * _NS
_TPW = _T // _NW
_LANES = 16
_CHUNKS = _D // _LANES


def _body(idx_hbm, wte_hbm, wpe_hbm, prm_hbm, out_hbm,
          idx_v, rows_v, wpe_v, prm_v,
          sem_g0, sem_g1, sem_g2, sem_g3,
          sem_h0, sem_h1, sem_h2, sem_h3, sem_w, sem_p, sem_st):
    gsems = (sem_g0, sem_g1, sem_g2, sem_g3)
    hsems = (sem_h0, sem_h1, sem_h2, sem_h3)
    wid = lax.axis_index("s") * _NC + lax.axis_index("c")
    t0 = wid * _TPW

    idx_cps = [pltpu.async_copy(idx_hbm.at[b, pl.ds(t0, _TPW)], idx_v.at[b],
                                gsems[b]) for b in range(_B)]
    wpe_cp = pltpu.async_copy(wpe_hbm.at[pl.ds(t0, _TPW)], wpe_v, sem_w)

    @pl.when(wid < _B)
    def _():
        pltpu.async_copy(prm_hbm, prm_v, sem_p).wait()
        pltpu.async_copy(prm_v, out_hbm.at[wid, pl.ds(0, _NPROMPT)],
                         sem_st)

    half = _TPW // 2
    gathers = []
    for b in range(_B):
        idx_cps[b].wait()
        gathers.append((
            pltpu.async_copy(wte_hbm.at[idx_v.at[b, pl.ds(0, half)]],
                             rows_v.at[b].at[pl.ds(0, half)], gsems[b]),
            pltpu.async_copy(wte_hbm.at[idx_v.at[b, pl.ds(half, half)]],
                             rows_v.at[b].at[pl.ds(half, half)], hsems[b]),
        ))
    wpe_cp.wait()

    stores = []
    for b in range(_B):
        buf = rows_v.at[b]

        def _add_row(i, _):
            for j in range(_CHUNKS):
                sl = pl.ds(j * _LANES, _LANES)
                plsc.addupdate(buf.at[i, sl], wpe_v[i, sl])
            return _

        gathers[b][0].wait()
        lax.fori_loop(0, half, _add_row, None)
        if b == _B - 1:
            stores.append(pltpu.async_copy(
                buf.at[pl.ds(0, half)],
                out_hbm.at[b, pl.ds(_NPROMPT + t0, half)], sem_st))
            gathers[b][1].wait()
            lax.fori_loop(half, _TPW, _add_row, None)
            stores.append(pltpu.async_copy(
                buf.at[pl.ds(half, half)],
                out_hbm.at[b, pl.ds(_NPROMPT + t0 + half, half)], sem_st))
        else:
            gathers[b][1].wait()
            lax.fori_loop(half, _TPW, _add_row, None)
            stores.append(pltpu.async_copy(
                buf, out_hbm.at[b, pl.ds(_NPROMPT + t0, _TPW)], sem_st))

    for cp in stores:
        cp.wait()

    @pl.when(wid < _B)
    def _():
        pltpu.make_async_copy(prm_v, out_hbm.at[wid, pl.ds(0, _NPROMPT)],
                              sem_st).wait()


@functools.partial(jax.jit, static_argnames=())
def kernel(idx, wte, wpe, reasoning_prompts):
    b, t = idx.shape
    assert (b, t) == (_B, _T)
    mesh = plsc.VectorSubcoreMesh(core_axis_name="c", subcore_axis_name="s")
    run = pl.kernel(
        _body,
        out_type=jax.ShapeDtypeStruct((_B, _NPROMPT + _T, _D), jnp.float32),
        mesh=mesh,
        scratch_types=[
            pltpu.VMEM((_B, _TPW), jnp.int32),
            pltpu.VMEM((_B, _TPW, _D), jnp.float32),
            pltpu.VMEM((_TPW, _D), jnp.float32),
            pltpu.VMEM((_NPROMPT, _D), jnp.float32),
            pltpu.SemaphoreType.DMA,
            pltpu.SemaphoreType.DMA,
            pltpu.SemaphoreType.DMA,
            pltpu.SemaphoreType.DMA,
            pltpu.SemaphoreType.DMA,
            pltpu.SemaphoreType.DMA,
            pltpu.SemaphoreType.DMA,
            pltpu.SemaphoreType.DMA,
            pltpu.SemaphoreType.DMA,
            pltpu.SemaphoreType.DMA,
            pltpu.SemaphoreType.DMA,
        ],
    )
    return run(idx.astype(jnp.int32), wte, wpe, reasoning_prompts)


# speedup vs baseline: 1.0163x; 1.0141x over previous
import functools

import jax
import jax.numpy as jnp
from jax import lax
from jax.experimental import pallas as pl
from jax.experimental.pallas import tpu as pltpu
from jax.experimental.pallas import tpu_sc as plsc

_B = 4
_T = 2048
_D = 128
_NPROMPT = 8

_INFO = plsc.get_sparse_core_info()
_NC = _INFO.num_cores
_NS = _INFO.num_subcores
_NW = _NC * _NS
_TPW = _T // _NW
_LANES = 16
_CHUNKS = _D // _LANES


def _body(idx_hbm, wte_hbm, wpe_hbm, prm_hbm, out_hbm,
          idx_v, rows_v, wpe_v, prm_v,
          sem_g0, sem_g1, sem_g2, sem_g3,
          sem_h0, sem_h1, sem_h2, sem_h3, sem_w, sem_p, sem_st):
    gsems = (sem_g0, sem_g1, sem_g2, sem_g3)
    hsems = (sem_h0, sem_h1, sem_h2, sem_h3)
    wid = lax.axis_index("s") * _NC + lax.axis_index("c")
    t0 = wid * _TPW

    idx_cps = [pltpu.async_copy(idx_hbm.at[b, pl.ds(t0, _TPW)], idx_v.at[b],
                                gsems[b]) for b in range(_B)]
    wpe_cp = pltpu.async_copy(wpe_hbm.at[pl.ds(t0, _TPW)], wpe_v, sem_w)

    @pl.when(wid < _B)
    def _():
        pltpu.async_copy(prm_hbm, prm_v, sem_p)

    half = _TPW // 2
    gathers = []
    for b in range(_B):
        idx_cps[b].wait()
        gathers.append((
            pltpu.async_copy(wte_hbm.at[idx_v.at[b, pl.ds(0, half)]],
                             rows_v.at[b].at[pl.ds(0, half)], gsems[b]),
            pltpu.async_copy(wte_hbm.at[idx_v.at[b, pl.ds(half, half)]],
                             rows_v.at[b].at[pl.ds(half, half)], hsems[b]),
        ))
    wpe_cp.wait()

    @pl.when(wid < _B)
    def _():
        pltpu.make_async_copy(prm_hbm, prm_v, sem_p).wait()
        pltpu.async_copy(prm_v, out_hbm.at[wid, pl.ds(0, _NPROMPT)],
                         sem_st)

    stores = []
    for b in range(_B):
        buf = rows_v.at[b]

        def _add_row(i, _):
            for j in range(_CHUNKS):
                sl = pl.ds(j * _LANES, _LANES)
                plsc.addupdate(buf.at[i, sl], wpe_v[i, sl])
            return _

        gathers[b][0].wait()
        lax.fori_loop(0, half, _add_row, None)
        if b == _B - 1:
            stores.append(pltpu.async_copy(
                buf.at[pl.ds(0, half)],
                out_hbm.at[b, pl.ds(_NPROMPT + t0, half)], sem_st))
            gathers[b][1].wait()
            lax.fori_loop(half, _TPW, _add_row, None)
            stores.append(pltpu.async_copy(
                buf.at[pl.ds(half, half)],
                out_hbm.at[b, pl.ds(_NPROMPT + t0 + half, half)], sem_st))
        else:
            gathers[b][1].wait()
            lax.fori_loop(half, _TPW, _add_row, None)
            stores.append(pltpu.async_copy(
                buf, out_hbm.at[b, pl.ds(_NPROMPT + t0, _TPW)], sem_st))

    for cp in stores:
        cp.wait()

    @pl.when(wid < _B)
    def _():
        pltpu.make_async_copy(prm_v, out_hbm.at[wid, pl.ds(0, _NPROMPT)],
                              sem_st).wait()


@functools.partial(jax.jit, static_argnames=())
def kernel(idx, wte, wpe, reasoning_prompts):
    b, t = idx.shape
    assert (b, t) == (_B, _T)
    mesh = plsc.VectorSubcoreMesh(core_axis_name="c", subcore_axis_name="s")
    run = pl.kernel(
        _body,
        out_type=jax.ShapeDtypeStruct((_B, _NPROMPT + _T, _D), jnp.float32),
        mesh=mesh,
        scratch_types=[
            pltpu.VMEM((_B, _TPW), jnp.int32),
            pltpu.VMEM((_B, _TPW, _D), jnp.float32),
            pltpu.VMEM((_TPW, _D), jnp.float32),
            pltpu.VMEM((_NPROMPT, _D), jnp.float32),
            pltpu.SemaphoreType.DMA,
            pltpu.SemaphoreType.DMA,
            pltpu.SemaphoreType.DMA,
            pltpu.SemaphoreType.DMA,
            pltpu.SemaphoreType.DMA,
            pltpu.SemaphoreType.DMA,
            pltpu.SemaphoreType.DMA,
            pltpu.SemaphoreType.DMA,
            pltpu.SemaphoreType.DMA,
            pltpu.SemaphoreType.DMA,
            pltpu.SemaphoreType.DMA,
        ],
    )
    return run(idx.astype(jnp.int32), wte, wpe, reasoning_prompts)
